# Initial kernel scaffold; baseline (speedup 1.0000x reference)
import jax
import jax.numpy as jnp
from jax.experimental import pallas as pl


def kernel(x, edge_index, W1, b1, W2, b2, W_out, b_out):
    raise NotImplementedError("write your pallas kernel here")


# speedup vs baseline: 1.2613x; 1.2613x over previous
import functools

import jax
import jax.numpy as jnp
from jax import lax
from jax.experimental import pallas as pl
from jax.experimental.pallas import tpu as pltpu

N = 10000
E = 160000
ESTEPS = 10
CE = E // ESTEPS
D = 256
_HI = jax.lax.Precision.HIGHEST

BM = 1000
GRID = N // BM


def _deg_body(ei_ref, out_ref, acc_ref):
    i = pl.program_id(0)

    @pl.when(i == 0)
    def _():
        acc_ref[...] = jnp.zeros_like(acc_ref)

    def body(e, _):
        v = ei_ref[1, e]
        acc_ref[pl.ds(v, 1), :] += 1.0
        return 0

    lax.fori_loop(0, CE, body, 0)

    @pl.when(i == ESTEPS - 1)
    def _():
        out_ref[...] = acc_ref[...]


def _deg_tc(ei):
    return pl.pallas_call(
        _deg_body,
        grid=(ESTEPS,),
        in_specs=[
            pl.BlockSpec((2, CE), lambda i: (0, i), memory_space=pltpu.SMEM),
        ],
        out_specs=pl.BlockSpec((N, 1), lambda i: (0, 0)),
        out_shape=jax.ShapeDtypeStruct((N, 1), jnp.float32),
        scratch_shapes=[pltpu.VMEM((N, 1), jnp.float32)],
    )(ei)


def _prop_body(g_ref, ei_ref, out_ref, acc_ref):
    i = pl.program_id(0)

    @pl.when(i == 0)
    def _():
        acc_ref[...] = g_ref[...]

    def body(e, _):
        s = ei_ref[0, e]
        d = ei_ref[1, e]
        acc_ref[pl.ds(d, 1), :] += g_ref[pl.ds(s, 1), :]
        return 0

    lax.fori_loop(0, CE, body, 0)

    @pl.when(i == ESTEPS - 1)
    def _():
        out_ref[...] = acc_ref[...]


def _prop_tc(g, ei):
    return pl.pallas_call(
        _prop_body,
        grid=(ESTEPS,),
        in_specs=[
            pl.BlockSpec((N, D), lambda i: (0, 0)),
            pl.BlockSpec((2, CE), lambda i: (0, i), memory_space=pltpu.SMEM),
        ],
        out_specs=pl.BlockSpec((N, D), lambda i: (0, 0)),
        out_shape=jax.ShapeDtypeStruct((N, D), jnp.float32),
        scratch_shapes=[pltpu.VMEM((N, D), jnp.float32)],
    )(g, ei)


def _tc1_body(deg_ref, x_ref, w_ref, g_ref, dinv_ref):
    deg = deg_ref[...] + 1.0
    dinv = lax.rsqrt(deg)
    h = jnp.dot(x_ref[...], w_ref[...], precision=_HI,
                preferred_element_type=jnp.float32)
    g_ref[...] = h * dinv
    dinv_ref[...] = dinv


def _tc1(deg, x, W1):
    return pl.pallas_call(
        _tc1_body,
        grid=(GRID,),
        in_specs=[
            pl.BlockSpec((BM, 1), lambda i: (i, 0)),
            pl.BlockSpec((BM, D), lambda i: (i, 0)),
            pl.BlockSpec((D, D), lambda i: (0, 0)),
        ],
        out_specs=[
            pl.BlockSpec((BM, D), lambda i: (i, 0)),
            pl.BlockSpec((BM, 1), lambda i: (i, 0)),
        ],
        out_shape=[
            jax.ShapeDtypeStruct((N, D), jnp.float32),
            jax.ShapeDtypeStruct((N, 1), jnp.float32),
        ],
    )(deg, x, W1)


def _tc2_body(s_ref, dinv_ref, b_ref, w_ref, g_ref):
    dinv = dinv_ref[...]
    h = jnp.maximum(s_ref[...] * dinv + b_ref[...], 0.0)
    g_ref[...] = jnp.dot(h, w_ref[...], precision=_HI,
                         preferred_element_type=jnp.float32) * dinv


def _tc2(s, dinv, b, W):
    return pl.pallas_call(
        _tc2_body,
        grid=(GRID,),
        in_specs=[
            pl.BlockSpec((BM, D), lambda i: (i, 0)),
            pl.BlockSpec((BM, 1), lambda i: (i, 0)),
            pl.BlockSpec((1, D), lambda i: (0, 0)),
            pl.BlockSpec((D, D), lambda i: (0, 0)),
        ],
        out_specs=pl.BlockSpec((BM, D), lambda i: (i, 0)),
        out_shape=jax.ShapeDtypeStruct((N, D), jnp.float32),
    )(s, dinv, b, W)


def _tc3_body(s_ref, dinv_ref, b_ref, wo_ref, bo_ref, o_ref):
    h = jnp.maximum(s_ref[...] * dinv_ref[...] + b_ref[...], 0.0)
    o_ref[...] = jnp.dot(h, wo_ref[...], precision=_HI,
                         preferred_element_type=jnp.float32) + bo_ref[...]


def _tc3(s, dinv, b, W_out, b_out):
    return pl.pallas_call(
        _tc3_body,
        grid=(GRID,),
        in_specs=[
            pl.BlockSpec((BM, D), lambda i: (i, 0)),
            pl.BlockSpec((BM, 1), lambda i: (i, 0)),
            pl.BlockSpec((1, D), lambda i: (0, 0)),
            pl.BlockSpec((D, 1), lambda i: (0, 0)),
            pl.BlockSpec((1, 1), lambda i: (0, 0)),
        ],
        out_specs=pl.BlockSpec((BM, 1), lambda i: (i, 0)),
        out_shape=jax.ShapeDtypeStruct((N, 1), jnp.float32),
    )(s, dinv, b, W_out, b_out)


def kernel(x, edge_index, W1, b1, W2, b2, W_out, b_out):
    ei = edge_index.astype(jnp.int32)

    deg = _deg_tc(ei)
    g1, dinv = _tc1(deg, x, W1)
    s1 = _prop_tc(g1, ei)
    g2 = _tc2(s1, dinv, b1.reshape(1, D), W2)
    s2 = _prop_tc(g2, ei)
    return _tc3(s2, dinv, b2.reshape(1, D), W_out, b_out.reshape(1, 1))


# speedup vs baseline: 1.2642x; 1.0023x over previous
import functools

import jax
import jax.numpy as jnp
from jax import lax
from jax.experimental import pallas as pl
from jax.experimental.pallas import tpu as pltpu

N = 10000
E = 160000
ESTEPS = 10
CE = E // ESTEPS
D = 256
_HI = None

BM = 1000
GRID = N // BM


def _deg_body(ei_ref, out_ref, acc_ref):
    i = pl.program_id(0)

    @pl.when(i == 0)
    def _():
        acc_ref[...] = jnp.zeros_like(acc_ref)

    def body(e, _):
        v = ei_ref[1, e]
        acc_ref[pl.ds(v, 1), :] += 1.0
        return 0

    lax.fori_loop(0, CE, body, 0)

    @pl.when(i == ESTEPS - 1)
    def _():
        out_ref[...] = acc_ref[...]


def _deg_tc(ei):
    return pl.pallas_call(
        _deg_body,
        grid=(ESTEPS,),
        in_specs=[
            pl.BlockSpec((2, CE), lambda i: (0, i), memory_space=pltpu.SMEM),
        ],
        out_specs=pl.BlockSpec((N, 1), lambda i: (0, 0)),
        out_shape=jax.ShapeDtypeStruct((N, 1), jnp.float32),
        scratch_shapes=[pltpu.VMEM((N, 1), jnp.float32)],
    )(ei)


def _prop_body(g_ref, ei_ref, out_ref, acc_ref):
    i = pl.program_id(0)

    @pl.when(i == 0)
    def _():
        acc_ref[...] = g_ref[...]

    def body(e, _):
        s = ei_ref[0, e]
        d = ei_ref[1, e]
        acc_ref[pl.ds(d, 1), :] += g_ref[pl.ds(s, 1), :]
        return 0

    lax.fori_loop(0, CE, body, 0)

    @pl.when(i == ESTEPS - 1)
    def _():
        out_ref[...] = acc_ref[...]


def _prop_tc(g, ei):
    return pl.pallas_call(
        _prop_body,
        grid=(ESTEPS,),
        in_specs=[
            pl.BlockSpec((N, D), lambda i: (0, 0)),
            pl.BlockSpec((2, CE), lambda i: (0, i), memory_space=pltpu.SMEM),
        ],
        out_specs=pl.BlockSpec((N, D), lambda i: (0, 0)),
        out_shape=jax.ShapeDtypeStruct((N, D), jnp.float32),
        scratch_shapes=[pltpu.VMEM((N, D), jnp.float32)],
    )(g, ei)


def _tc1_body(deg_ref, x_ref, w_ref, g_ref, dinv_ref):
    deg = deg_ref[...] + 1.0
    dinv = lax.rsqrt(deg)
    h = jnp.dot(x_ref[...], w_ref[...], precision=_HI,
                preferred_element_type=jnp.float32)
    g_ref[...] = h * dinv
    dinv_ref[...] = dinv


def _tc1(deg, x, W1):
    return pl.pallas_call(
        _tc1_body,
        grid=(GRID,),
        in_specs=[
            pl.BlockSpec((BM, 1), lambda i: (i, 0)),
            pl.BlockSpec((BM, D), lambda i: (i, 0)),
            pl.BlockSpec((D, D), lambda i: (0, 0)),
        ],
        out_specs=[
            pl.BlockSpec((BM, D), lambda i: (i, 0)),
            pl.BlockSpec((BM, 1), lambda i: (i, 0)),
        ],
        out_shape=[
            jax.ShapeDtypeStruct((N, D), jnp.float32),
            jax.ShapeDtypeStruct((N, 1), jnp.float32),
        ],
    )(deg, x, W1)


def _tc2_body(s_ref, dinv_ref, b_ref, w_ref, g_ref):
    dinv = dinv_ref[...]
    h = jnp.maximum(s_ref[...] * dinv + b_ref[...], 0.0)
    g_ref[...] = jnp.dot(h, w_ref[...], precision=_HI,
                         preferred_element_type=jnp.float32) * dinv


def _tc2(s, dinv, b, W):
    return pl.pallas_call(
        _tc2_body,
        grid=(GRID,),
        in_specs=[
            pl.BlockSpec((BM, D), lambda i: (i, 0)),
            pl.BlockSpec((BM, 1), lambda i: (i, 0)),
            pl.BlockSpec((1, D), lambda i: (0, 0)),
            pl.BlockSpec((D, D), lambda i: (0, 0)),
        ],
        out_specs=pl.BlockSpec((BM, D), lambda i: (i, 0)),
        out_shape=jax.ShapeDtypeStruct((N, D), jnp.float32),
    )(s, dinv, b, W)


def _tc3_body(s_ref, dinv_ref, b_ref, wo_ref, bo_ref, o_ref):
    h = jnp.maximum(s_ref[...] * dinv_ref[...] + b_ref[...], 0.0)
    o_ref[...] = jnp.dot(h, wo_ref[...], precision=_HI,
                         preferred_element_type=jnp.float32) + bo_ref[...]


def _tc3(s, dinv, b, W_out, b_out):
    return pl.pallas_call(
        _tc3_body,
        grid=(GRID,),
        in_specs=[
            pl.BlockSpec((BM, D), lambda i: (i, 0)),
            pl.BlockSpec((BM, 1), lambda i: (i, 0)),
            pl.BlockSpec((1, D), lambda i: (0, 0)),
            pl.BlockSpec((D, 1), lambda i: (0, 0)),
            pl.BlockSpec((1, 1), lambda i: (0, 0)),
        ],
        out_specs=pl.BlockSpec((BM, 1), lambda i: (i, 0)),
        out_shape=jax.ShapeDtypeStruct((N, 1), jnp.float32),
    )(s, dinv, b, W_out, b_out)


def kernel(x, edge_index, W1, b1, W2, b2, W_out, b_out):
    ei = edge_index.astype(jnp.int32)

    deg = _deg_tc(ei)
    g1, dinv = _tc1(deg, x, W1)
    s1 = _prop_tc(g1, ei)
    g2 = _tc2(s1, dinv, b1.reshape(1, D), W2)
    s2 = _prop_tc(g2, ei)
    return _tc3(s2, dinv, b2.reshape(1, D), W_out, b_out.reshape(1, 1))
